# baseline (speedup 1.0000x reference)
import numpy as np
import jax
import jax.numpy as jnp
from jax.experimental import pallas as pl
from jax.experimental.pallas import tpu as pltpu

_B, _K, _N, _M = 256, 8, 64, 32
_P = _N * (_N - 1) // 2
_R = _B * _K
_F2 = _N * _M
_T = 512

_ROWS, _COLS = np.triu_indices(_N, k=1)
_OFF = np.concatenate([[0], np.cumsum(np.arange(_N - 1, 0, -1))]).astype(np.int64)

_RT_NP = np.zeros((_N, _P), np.float32)
_RT_NP[_ROWS, np.arange(_P)] += 1.0
_RT_NP[_COLS, np.arange(_P)] += 1.0

def _prep_kernel(w_ref, rt_ref, w2_ref, clin_ref):
    wa = w_ref[0:_P, :]
    wo = w_ref[_P:2 * _P, :]
    wx = w_ref[2 * _P:3 * _P, :]
    a = wa - wo - 2.0 * wx
    c = wo + wx

    w2_ref[:, :] = jnp.zeros((_N * _N, _M), jnp.float32)
    for i in range(_N - 1):
        cnt = _N - 1 - i
        w2_ref[i * _N + i + 1:i * _N + _N, :] = a[int(_OFF[i]):int(_OFF[i]) + cnt, :]
    clin_ref[:, :] = jnp.dot(rt_ref[:, :], c, preferred_element_type=jnp.float32)


def _main_kernel(x_ref, w2b_ref, clin_ref, b_ref, out_ref):
    x = x_ref[:, :]
    y = jnp.dot(x, w2b_ref[:, :], preferred_element_type=jnp.float32)
    xrep = jnp.broadcast_to(x[:, :, None], (_T, _N, _M)).reshape(_T, _F2)
    z = y * xrep
    while z.shape[1] > _M:
        h = z.shape[1] // 2
        z = z[:, :h] + z[:, h:]
    out_ref[:, :] = z + jnp.dot(x, clin_ref[:, :],
                                preferred_element_type=jnp.float32) + b_ref[:, :]


def kernel(inputs, W, b):
    x2d = inputs.reshape(_R, _N)
    b2d = b.reshape(1, _M)
    rt = jnp.asarray(_RT_NP)
    w2, clin = pl.pallas_call(
        _prep_kernel,
        out_shape=[jax.ShapeDtypeStruct((_N * _N, _M), jnp.float32),
                   jax.ShapeDtypeStruct((_N, _M), jnp.float32)],
    )(W, rt)
    w2big = w2.reshape(_N, _F2)
    out = pl.pallas_call(
        _main_kernel,
        grid=(_R // _T,),
        in_specs=[pl.BlockSpec((_T, _N), lambda r: (r, 0)),
                  pl.BlockSpec((_N, _F2), lambda r: (0, 0)),
                  pl.BlockSpec((_N, _M), lambda r: (0, 0)),
                  pl.BlockSpec((1, _M), lambda r: (0, 0))],
        out_specs=pl.BlockSpec((_T, _M), lambda r: (r, 0)),
        out_shape=jax.ShapeDtypeStruct((_R, _M), jnp.float32),
    )(x2d, w2big, clin, b2d)
    return out.reshape(_B, _K, _M)


# speedup vs baseline: 1.8013x; 1.8013x over previous
import numpy as np
import jax
import jax.numpy as jnp
from jax.experimental import pallas as pl
from jax.experimental.pallas import tpu as pltpu

_B, _K, _N, _M = 256, 8, 64, 32
_P = _N * (_N - 1) // 2
_R = _B * _K
_F2 = _N * _M
_T = 512

_ROWS, _COLS = np.triu_indices(_N, k=1)
_OFF = np.concatenate([[0], np.cumsum(np.arange(_N - 1, 0, -1))]).astype(np.int64)

_RT_NP = np.zeros((_N, _P), np.float32)
_RT_NP[_ROWS, np.arange(_P)] += 1.0
_RT_NP[_COLS, np.arange(_P)] += 1.0

def _prep_kernel(w_ref, rt_ref, w2_ref, clin_ref):
    wa = w_ref[0:_P, :]
    wo = w_ref[_P:2 * _P, :]
    wx = w_ref[2 * _P:3 * _P, :]
    a = wa - wo - 2.0 * wx
    c = wo + wx

    w2_ref[:, :] = jnp.zeros((_N * _N, _M), jnp.float32)
    for i in range(_N - 1):
        cnt = _N - 1 - i
        w2_ref[i * _N + i + 1:i * _N + _N, :] = a[int(_OFF[i]):int(_OFF[i]) + cnt, :]
    clin_ref[:, :] = jnp.dot(rt_ref[:, :], c, preferred_element_type=jnp.float32)


def _main_kernel(x_ref, w2b_ref, clin_ref, b_ref, out_ref):
    x = x_ref[:, :]
    y = jnp.dot(x, w2b_ref[:, :], preferred_element_type=jnp.float32)
    idx = jax.lax.broadcasted_iota(jnp.int32, (_T, _F2), 1) // _M
    xrep = jnp.take_along_axis(x, idx, axis=1)
    z = y * xrep
    while z.shape[1] > _M:
        h = z.shape[1] // 2
        z = z[:, :h] + z[:, h:]
    out_ref[:, :] = z + jnp.dot(x, clin_ref[:, :],
                                preferred_element_type=jnp.float32) + b_ref[:, :]


def kernel(inputs, W, b):
    x2d = inputs.reshape(_R, _N)
    b2d = b.reshape(1, _M)
    rt = jnp.asarray(_RT_NP)
    w2, clin = pl.pallas_call(
        _prep_kernel,
        out_shape=[jax.ShapeDtypeStruct((_N * _N, _M), jnp.float32),
                   jax.ShapeDtypeStruct((_N, _M), jnp.float32)],
    )(W, rt)
    w2big = w2.reshape(_N, _F2)
    out = pl.pallas_call(
        _main_kernel,
        grid=(_R // _T,),
        in_specs=[pl.BlockSpec((_T, _N), lambda r: (r, 0)),
                  pl.BlockSpec((_N, _F2), lambda r: (0, 0)),
                  pl.BlockSpec((_N, _M), lambda r: (0, 0)),
                  pl.BlockSpec((1, _M), lambda r: (0, 0))],
        out_specs=pl.BlockSpec((_T, _M), lambda r: (r, 0)),
        out_shape=jax.ShapeDtypeStruct((_R, _M), jnp.float32),
    )(x2d, w2big, clin, b2d)
    return out.reshape(_B, _K, _M)


# speedup vs baseline: 2.3594x; 1.3098x over previous
import numpy as np
import jax
import jax.numpy as jnp
from jax.experimental import pallas as pl
from jax.experimental.pallas import tpu as pltpu

_B, _K, _N, _M = 256, 8, 64, 32
_P = _N * (_N - 1) // 2
_R = _B * _K
_F2 = _N * _M
_T = 512

_ROWS, _COLS = np.triu_indices(_N, k=1)
_OFF = np.concatenate([[0], np.cumsum(np.arange(_N - 1, 0, -1))]).astype(np.int64)

_RT_NP = np.zeros((_N, _P), np.float32)
_RT_NP[_ROWS, np.arange(_P)] += 1.0
_RT_NP[_COLS, np.arange(_P)] += 1.0

def _prep_kernel(w_ref, rt_ref, w2_ref, clin_ref):
    wa = w_ref[0:_P, :]
    wo = w_ref[_P:2 * _P, :]
    wx = w_ref[2 * _P:3 * _P, :]
    a = wa - wo - 2.0 * wx
    c = wo + wx

    w2_ref[:, :] = jnp.zeros((_N * _N, _M), jnp.float32)
    for i in range(_N - 1):
        cnt = _N - 1 - i
        w2_ref[i * _N + i + 1:i * _N + _N, :] = a[int(_OFF[i]):int(_OFF[i]) + cnt, :]
    clin_ref[:, :] = jnp.dot(rt_ref[:, :], c, preferred_element_type=jnp.float32)


def _main_kernel(x_ref, w2b_ref, clin_ref, b_ref, out_ref):
    x = x_ref[:, :]
    y = jnp.dot(x, w2b_ref[:, :], preferred_element_type=jnp.float32)
    lane = jax.lax.broadcasted_iota(jnp.int32, (_N, _F2), 1)
    row = jax.lax.broadcasted_iota(jnp.int32, (_N, _F2), 0)
    e = (lane // _M == row).astype(jnp.float32)
    xrep = jnp.dot(x, e, preferred_element_type=jnp.float32)
    z = y * xrep
    while z.shape[1] > _M:
        h = z.shape[1] // 2
        z = z[:, :h] + z[:, h:]
    out_ref[:, :] = z + jnp.dot(x, clin_ref[:, :],
                                preferred_element_type=jnp.float32) + b_ref[:, :]


def kernel(inputs, W, b):
    x2d = inputs.reshape(_R, _N)
    b2d = b.reshape(1, _M)
    rt = jnp.asarray(_RT_NP)
    w2, clin = pl.pallas_call(
        _prep_kernel,
        out_shape=[jax.ShapeDtypeStruct((_N * _N, _M), jnp.float32),
                   jax.ShapeDtypeStruct((_N, _M), jnp.float32)],
    )(W, rt)
    w2big = w2.reshape(_N, _F2)
    out = pl.pallas_call(
        _main_kernel,
        grid=(_R // _T,),
        in_specs=[pl.BlockSpec((_T, _N), lambda r: (r, 0)),
                  pl.BlockSpec((_N, _F2), lambda r: (0, 0)),
                  pl.BlockSpec((_N, _M), lambda r: (0, 0)),
                  pl.BlockSpec((1, _M), lambda r: (0, 0))],
        out_specs=pl.BlockSpec((_T, _M), lambda r: (r, 0)),
        out_shape=jax.ShapeDtypeStruct((_R, _M), jnp.float32),
    )(x2d, w2big, clin, b2d)
    return out.reshape(_B, _K, _M)
